# baseline (speedup 1.0000x reference)
import functools

import jax
import jax.numpy as jnp
from jax import lax
from jax.experimental import pallas as pl
from jax.experimental.pallas import tpu as pltpu
from jax.experimental.pallas import tpu_sc as plsc

_N = 10000
_E = 320000
_D = 128
_T = 12
_NC = 2
_NS = 16
_NW = _NC * _NS
_EPW = _E // _NW
_CHB = 100
_NCH = _EPW // _CHB
_NPAD = 10240
_RPT = _NPAD // _NS
_ZR = 80
_BLK = 2000
_NBLK = _N // _BLK
_EPS = 1e-5


def _sc_agg_body(h_hbm, src_hbm, dst_hbm, out_hbm,
                 acc, sidx, didx, rows, isems, rsems, ssems):
    c = lax.axis_index("c")
    s = lax.axis_index("s")
    wid = c * _NS + s

    def idx_start(j, p):
        pltpu.async_copy(src_hbm.at[wid, j], sidx[p], isems[p])
        pltpu.async_copy(dst_hbm.at[wid, j], didx[p], isems[p])

    def idx_wait(j, p):
        pltpu.make_async_copy(src_hbm.at[wid, j], sidx[p], isems[p]).wait()
        pltpu.make_async_copy(dst_hbm.at[wid, j], didx[p], isems[p]).wait()

    def gather_start(j, p, r):
        pltpu.async_copy(h_hbm.at[sidx[p].at[0]], rows[r], rsems[r])

    def gather_wait(p, r):
        pltpu.make_async_copy(h_hbm.at[sidx[p].at[0]], rows[r], rsems[r]).wait()

    _HH = _CHB // 2

    def scatter_start(p, r):
        pltpu.async_copy(rows[r].at[pl.ds(0, _HH)],
                         acc.at[didx[p].at[0]], ssems[2 * r], add=True)
        pltpu.async_copy(rows[r].at[pl.ds(_HH, _HH)],
                         acc.at[didx[p].at[1]], ssems[2 * r + 1], add=True)

    def scatter_wait(p, r):
        pltpu.make_async_copy(rows[r].at[pl.ds(0, _HH)],
                              acc.at[didx[p].at[0]], ssems[2 * r]).wait()
        pltpu.make_async_copy(rows[r].at[pl.ds(_HH, _HH)],
                              acc.at[didx[p].at[1]], ssems[2 * r + 1]).wait()

    for p in range(3):
        idx_start(p, p)

    z16 = jnp.zeros((16,), jnp.float32)

    def zstep(i, carry):
        rows[1][i // 8, pl.ds((i % 8) * 16, 16)] = z16
        return carry

    lax.fori_loop(0, _ZR * 8, zstep, 0)
    idx_wait(0, 0)
    gather_start(0, 0, 0)

    def zcopy(k, carry):
        pltpu.sync_copy(rows[1].at[pl.ds(0, _ZR)],
                        acc.at[pl.ds(s * _RPT + k * _ZR, _ZR)])
        return carry

    lax.fori_loop(0, _RPT // _ZR, zcopy, 0)
    plsc.subcore_barrier()

    def step(jj, carry):
        j0 = jj * 4

        def chunk(u):
            j = j0 + u
            p = u % 4
            pn = (u + 1) % 4
            pf = (u + 3) % 4
            r = u % 2
            rn = (u + 1) % 2

            if u == 3:
                @pl.when(j + 1 < _NCH)
                def _():
                    idx_wait(j + 1, pn)
            else:
                idx_wait(j + 1, pn)
            gather_wait(p, r)
            if u == 0:
                @pl.when(j > 0)
                def _():
                    scatter_wait(pf, rn)
            else:
                scatter_wait(pf, rn)

            @pl.when(j + 3 < _NCH)
            def _():
                idx_start(j + 3, pf)

            if u == 3:
                @pl.when(j + 1 < _NCH)
                def _():
                    gather_start(j + 1, pn, rn)
            else:
                gather_start(j + 1, pn, rn)
            scatter_start(p, r)

        for u in range(4):
            chunk(u)
        return carry

    lax.fori_loop(0, _NCH // 4, step, 0)
    scatter_wait(3, 1)
    plsc.subcore_barrier()

    pltpu.sync_copy(acc.at[pl.ds(s * _RPT, _RPT)],
                    out_hbm.at[c, pl.ds(s * _RPT, _RPT)])


@functools.cache
def _build_sc_agg():
    return pl.kernel(
        _sc_agg_body,
        out_type=jax.ShapeDtypeStruct((_NC, _NPAD, _D), jnp.float32),
        mesh=plsc.VectorSubcoreMesh(core_axis_name="c", subcore_axis_name="s",
                                    num_cores=_NC, num_subcores=_NS),
        scratch_types=[
            pltpu.VMEM_SHARED((_NPAD, _D), jnp.float32),
            [pltpu.VMEM((1, _CHB), jnp.int32) for _ in range(4)],
            [pltpu.VMEM((2, _CHB // 2), jnp.int32) for _ in range(4)],
            [pltpu.VMEM((_CHB, _D), jnp.float32) for _ in range(2)],
            [pltpu.SemaphoreType.DMA for _ in range(4)],
            [pltpu.SemaphoreType.DMA for _ in range(2)],
            [pltpu.SemaphoreType.DMA for _ in range(4)],
        ],
    )


def _mm_body(h_ref, w_ref, out_ref):
    out_ref[...] = jnp.dot(h_ref[...], w_ref[...],
                           preferred_element_type=jnp.float32)


_tc_mm = pl.pallas_call(
    _mm_body,
    grid=(_NBLK,),
    in_specs=[
        pl.BlockSpec((_BLK, _D), lambda i: (i, 0)),
        pl.BlockSpec((_D, _D), lambda i: (0, 0)),
    ],
    out_specs=pl.BlockSpec((_BLK, _D), lambda i: (i, 0)),
    out_shape=jax.ShapeDtypeStruct((_N, _D), jnp.float32),
)


def _layer_pre(i, deg_ref, hs_ref, agg_ref, wn_ref, b_ref, stats_s, pre_s):
    dinv = 1.0 / jnp.maximum(deg_ref[...], 1).astype(jnp.float32)
    a = (agg_ref[0] + agg_ref[1]) * dinv
    pre = (hs_ref[...]
           + jnp.dot(a, wn_ref[...], preferred_element_type=jnp.float32)
           + b_ref[...])
    pre_s[pl.ds(i * _BLK, _BLK), :] = pre
    st = jnp.concatenate(
        [jnp.sum(pre, axis=0)[None], jnp.sum(pre * pre, axis=0)[None]], axis=0)

    @pl.when(i == 0)
    def _():
        stats_s[...] = st

    @pl.when(i > 0)
    def _():
        stats_s[...] += st


def _bn_scale(stats_s, g_ref):
    m = stats_s[0:1, :] * (1.0 / _N)
    v = stats_s[1:2, :] * (1.0 / _N) - m * m
    return m, lax.rsqrt(v + _EPS) * g_ref[...]


def _tc_layer1_body(deg_ref, hs_ref, agg_ref, wn_ref, b_ref,
                    g_ref, be_ref, out_ref, stats_s, pre_s):
    p = pl.program_id(0)
    i = pl.program_id(1)

    @pl.when(p == 0)
    def _():
        _layer_pre(i, deg_ref, hs_ref, agg_ref, wn_ref, b_ref,
                   stats_s, pre_s)
        out_ref[...] = jnp.zeros((_BLK, _D), jnp.float32)

    @pl.when(p == 1)
    def _():
        m, scale = _bn_scale(stats_s, g_ref)
        pre = pre_s[pl.ds(i * _BLK, _BLK), :]
        out_ref[...] = jnp.maximum((pre - m) * scale + be_ref[...], 0.0)


_tc_layer1 = pl.pallas_call(
    _tc_layer1_body,
    grid=(2, _NBLK),
    in_specs=[
        pl.BlockSpec((_BLK, 1), lambda p, i: (i * (1 - p), 0)),
        pl.BlockSpec((_BLK, _D), lambda p, i: (i * (1 - p), 0)),
        pl.BlockSpec((_NC, _BLK, _D), lambda p, i: (0, i * (1 - p), 0)),
        pl.BlockSpec((_D, _D), lambda p, i: (0, 0)),
        pl.BlockSpec((1, _D), lambda p, i: (0, 0)),
        pl.BlockSpec((1, _D), lambda p, i: (0, 0)),
        pl.BlockSpec((1, _D), lambda p, i: (0, 0)),
    ],
    out_specs=pl.BlockSpec((_BLK, _D), lambda p, i: (i * p, 0)),
    out_shape=jax.ShapeDtypeStruct((_N, _D), jnp.float32),
    scratch_shapes=[
        pltpu.VMEM((2, _D), jnp.float32),
        pltpu.VMEM((_N, _D), jnp.float32),
    ],
)


def _tc_layer2_body(deg_ref, hs_ref, agg_ref, wn_ref, b_ref,
                    g_ref, be_ref, wd_ref, bd_ref, out_ref,
                    stats_s, pre_s, pool_s):
    p = pl.program_id(0)
    i = pl.program_id(1)

    @pl.when(p == 0)
    def _():
        _layer_pre(i, deg_ref, hs_ref, agg_ref, wn_ref, b_ref,
                   stats_s, pre_s)

    @pl.when(p == 1)
    def _():
        m, scale = _bn_scale(stats_s, g_ref)
        pre = pre_s[pl.ds(i * _BLK, _BLK), :]
        h2 = jnp.maximum((pre - m) * scale + be_ref[...], 0.0)
        cs = jnp.sum(h2, axis=0)[None]

        @pl.when(i == 0)
        def _():
            pool_s[...] = cs

        @pl.when(i > 0)
        def _():
            pool_s[...] += cs

        @pl.when(i == _NBLK - 1)
        def _():
            out_ref[...] = (jnp.dot(pool_s[...] * (1.0 / _N), wd_ref[...],
                                    preferred_element_type=jnp.float32)
                            + bd_ref[...])


_tc_layer2 = pl.pallas_call(
    _tc_layer2_body,
    grid=(2, _NBLK),
    in_specs=[
        pl.BlockSpec((_BLK, 1), lambda p, i: (i * (1 - p), 0)),
        pl.BlockSpec((_BLK, _D), lambda p, i: (i * (1 - p), 0)),
        pl.BlockSpec((_NC, _BLK, _D), lambda p, i: (0, i * (1 - p), 0)),
        pl.BlockSpec((_D, _D), lambda p, i: (0, 0)),
        pl.BlockSpec((1, _D), lambda p, i: (0, 0)),
        pl.BlockSpec((1, _D), lambda p, i: (0, 0)),
        pl.BlockSpec((1, _D), lambda p, i: (0, 0)),
        pl.BlockSpec((_D, _T), lambda p, i: (0, 0)),
        pl.BlockSpec((1, _T), lambda p, i: (0, 0)),
    ],
    out_specs=pl.BlockSpec((1, _T), lambda p, i: (0, 0)),
    out_shape=jax.ShapeDtypeStruct((1, _T), jnp.float32),
    scratch_shapes=[
        pltpu.VMEM((2, _D), jnp.float32),
        pltpu.VMEM((_N, _D), jnp.float32),
        pltpu.VMEM((1, _D), jnp.float32),
    ],
)


def kernel(x, adjacency_list, degree_list,
           W1s, W1n, b1, g1, be1, W2s, W2n, b2, g2, be2, Wd, bd):
    src = adjacency_list[0].reshape(_NW, _NCH, 1, _CHB)
    dst = adjacency_list[1].reshape(_NW, _NCH, 2, _CHB // 2)
    deg = degree_list.reshape(_N, 1)

    sc_agg = _build_sc_agg()
    xs = _tc_mm(x, W1s)
    agg1 = sc_agg(x, src, dst)
    h1 = _tc_layer1(deg, xs, agg1, W1n, b1.reshape(1, _D),
                    g1.reshape(1, _D), be1.reshape(1, _D))

    hs2 = _tc_mm(h1, W2s)
    agg2 = sc_agg(h1, src, dst)
    out = _tc_layer2(deg, hs2, agg2, W2n, b2.reshape(1, _D),
                     g2.reshape(1, _D), be2.reshape(1, _D),
                     Wd, bd.reshape(1, _T))
    return out.reshape(_T)


# speedup vs baseline: 1.0117x; 1.0117x over previous
import functools

import jax
import jax.numpy as jnp
from jax import lax
from jax.experimental import pallas as pl
from jax.experimental.pallas import tpu as pltpu
from jax.experimental.pallas import tpu_sc as plsc

_N = 10000
_E = 320000
_D = 128
_T = 12
_NC = 2
_NS = 16
_NW = _NC * _NS
_EPW = _E // _NW
_CHB = 100
_NCH = _EPW // _CHB
_NPAD = 10240
_RPT = _NPAD // _NS
_ZR = 80
_BLK = 2000
_NBLK = _N // _BLK
_EPS = 1e-5


def _sc_agg_body(h_hbm, src_hbm, dst_hbm, out_hbm,
                 acc, sidx, didx, rows, isems, rsems, ssems):
    c = lax.axis_index("c")
    s = lax.axis_index("s")
    wid = c * _NS + s

    def idx_start(j, p):
        pltpu.async_copy(src_hbm.at[wid, j], sidx[p], isems[p])
        pltpu.async_copy(dst_hbm.at[wid, j], didx[p], isems[p])

    def idx_wait(j, p):
        pltpu.make_async_copy(src_hbm.at[wid, j], sidx[p], isems[p]).wait()
        pltpu.make_async_copy(dst_hbm.at[wid, j], didx[p], isems[p]).wait()

    def gather_start(j, p, r):
        pltpu.async_copy(h_hbm.at[sidx[p].at[0]], rows[r], rsems[r])

    def gather_wait(p, r):
        pltpu.make_async_copy(h_hbm.at[sidx[p].at[0]], rows[r], rsems[r]).wait()

    def scatter_start(p, r):
        pltpu.async_copy(rows[r], acc.at[didx[p].at[0]], ssems[r], add=True)

    def scatter_wait(p, r):
        pltpu.make_async_copy(rows[r], acc.at[didx[p].at[0]], ssems[r]).wait()

    for p in range(3):
        idx_start(p, p)

    z16 = jnp.zeros((16,), jnp.float32)

    def zstep(i, carry):
        rows[1][i // 8, pl.ds((i % 8) * 16, 16)] = z16
        return carry

    lax.fori_loop(0, _ZR * 8, zstep, 0)
    idx_wait(0, 0)
    gather_start(0, 0, 0)

    def zcopy(k, carry):
        pltpu.sync_copy(rows[1].at[pl.ds(0, _ZR)],
                        acc.at[pl.ds(s * _RPT + k * _ZR, _ZR)])
        return carry

    lax.fori_loop(0, _RPT // _ZR, zcopy, 0)
    plsc.subcore_barrier()

    def step(jj, carry):
        j0 = jj * 4

        def chunk(u):
            j = j0 + u
            p = u % 4
            pn = (u + 1) % 4
            pf = (u + 3) % 4
            r = u % 2
            rn = (u + 1) % 2

            if u == 3:
                @pl.when(j + 1 < _NCH)
                def _():
                    idx_wait(j + 1, pn)
            else:
                idx_wait(j + 1, pn)
            gather_wait(p, r)
            if u == 0:
                @pl.when(j > 0)
                def _():
                    scatter_wait(pf, rn)
            else:
                scatter_wait(pf, rn)

            @pl.when(j + 3 < _NCH)
            def _():
                idx_start(j + 3, pf)

            if u == 3:
                @pl.when(j + 1 < _NCH)
                def _():
                    gather_start(j + 1, pn, rn)
            else:
                gather_start(j + 1, pn, rn)
            scatter_start(p, r)

        for u in range(4):
            chunk(u)
        return carry

    lax.fori_loop(0, _NCH // 4, step, 0)
    scatter_wait(3, 1)
    plsc.subcore_barrier()

    pltpu.sync_copy(acc.at[pl.ds(s * _RPT, _RPT)],
                    out_hbm.at[c, pl.ds(s * _RPT, _RPT)])


@functools.cache
def _build_sc_agg():
    return pl.kernel(
        _sc_agg_body,
        out_type=jax.ShapeDtypeStruct((_NC, _NPAD, _D), jnp.float32),
        mesh=plsc.VectorSubcoreMesh(core_axis_name="c", subcore_axis_name="s",
                                    num_cores=_NC, num_subcores=_NS),
        scratch_types=[
            pltpu.VMEM_SHARED((_NPAD, _D), jnp.float32),
            [pltpu.VMEM((1, _CHB), jnp.int32) for _ in range(4)],
            [pltpu.VMEM((1, _CHB), jnp.int32) for _ in range(4)],
            [pltpu.VMEM((_CHB, _D), jnp.float32) for _ in range(2)],
            [pltpu.SemaphoreType.DMA for _ in range(4)],
            [pltpu.SemaphoreType.DMA for _ in range(2)],
            [pltpu.SemaphoreType.DMA for _ in range(2)],
        ],
    )


def _mm_body(h_ref, w_ref, out_ref):
    out_ref[...] = jnp.dot(h_ref[...], w_ref[...],
                           preferred_element_type=jnp.float32)


_tc_mm = pl.pallas_call(
    _mm_body,
    grid=(_NBLK,),
    in_specs=[
        pl.BlockSpec((_BLK, _D), lambda i: (i, 0)),
        pl.BlockSpec((_D, _D), lambda i: (0, 0)),
    ],
    out_specs=pl.BlockSpec((_BLK, _D), lambda i: (i, 0)),
    out_shape=jax.ShapeDtypeStruct((_N, _D), jnp.float32),
)


def _layer_pre(i, deg_ref, hs_ref, agg_ref, wn_ref, b_ref, stats_s, pre_s):
    dinv = 1.0 / jnp.maximum(deg_ref[...], 1).astype(jnp.float32)
    a = (agg_ref[0] + agg_ref[1]) * dinv
    pre = (hs_ref[...]
           + jnp.dot(a, wn_ref[...], preferred_element_type=jnp.float32)
           + b_ref[...])
    pre_s[pl.ds(i * _BLK, _BLK), :] = pre
    st = jnp.concatenate(
        [jnp.sum(pre, axis=0)[None], jnp.sum(pre * pre, axis=0)[None]], axis=0)

    @pl.when(i == 0)
    def _():
        stats_s[...] = st

    @pl.when(i > 0)
    def _():
        stats_s[...] += st


def _bn_scale(stats_s, g_ref):
    m = stats_s[0:1, :] * (1.0 / _N)
    v = stats_s[1:2, :] * (1.0 / _N) - m * m
    return m, lax.rsqrt(v + _EPS) * g_ref[...]


def _tc_layer1_body(deg_ref, hs_ref, agg_ref, wn_ref, b_ref,
                    g_ref, be_ref, out_ref, stats_s, pre_s):
    p = pl.program_id(0)
    i = pl.program_id(1)

    @pl.when(p == 0)
    def _():
        _layer_pre(i, deg_ref, hs_ref, agg_ref, wn_ref, b_ref,
                   stats_s, pre_s)
        out_ref[...] = jnp.zeros((_BLK, _D), jnp.float32)

    @pl.when(p == 1)
    def _():
        m, scale = _bn_scale(stats_s, g_ref)
        pre = pre_s[pl.ds(i * _BLK, _BLK), :]
        out_ref[...] = jnp.maximum((pre - m) * scale + be_ref[...], 0.0)


_tc_layer1 = pl.pallas_call(
    _tc_layer1_body,
    grid=(2, _NBLK),
    in_specs=[
        pl.BlockSpec((_BLK, 1), lambda p, i: (i * (1 - p), 0)),
        pl.BlockSpec((_BLK, _D), lambda p, i: (i * (1 - p), 0)),
        pl.BlockSpec((_NC, _BLK, _D), lambda p, i: (0, i * (1 - p), 0)),
        pl.BlockSpec((_D, _D), lambda p, i: (0, 0)),
        pl.BlockSpec((1, _D), lambda p, i: (0, 0)),
        pl.BlockSpec((1, _D), lambda p, i: (0, 0)),
        pl.BlockSpec((1, _D), lambda p, i: (0, 0)),
    ],
    out_specs=pl.BlockSpec((_BLK, _D), lambda p, i: (i * p, 0)),
    out_shape=jax.ShapeDtypeStruct((_N, _D), jnp.float32),
    scratch_shapes=[
        pltpu.VMEM((2, _D), jnp.float32),
        pltpu.VMEM((_N, _D), jnp.float32),
    ],
)


def _tc_layer2_body(deg_ref, hs_ref, agg_ref, wn_ref, b_ref,
                    g_ref, be_ref, wd_ref, bd_ref, out_ref,
                    stats_s, pre_s, pool_s):
    p = pl.program_id(0)
    i = pl.program_id(1)

    @pl.when(p == 0)
    def _():
        _layer_pre(i, deg_ref, hs_ref, agg_ref, wn_ref, b_ref,
                   stats_s, pre_s)

    @pl.when(p == 1)
    def _():
        m, scale = _bn_scale(stats_s, g_ref)
        pre = pre_s[pl.ds(i * _BLK, _BLK), :]
        h2 = jnp.maximum((pre - m) * scale + be_ref[...], 0.0)
        cs = jnp.sum(h2, axis=0)[None]

        @pl.when(i == 0)
        def _():
            pool_s[...] = cs

        @pl.when(i > 0)
        def _():
            pool_s[...] += cs

        @pl.when(i == _NBLK - 1)
        def _():
            out_ref[...] = (jnp.dot(pool_s[...] * (1.0 / _N), wd_ref[...],
                                    preferred_element_type=jnp.float32)
                            + bd_ref[...])


_tc_layer2 = pl.pallas_call(
    _tc_layer2_body,
    grid=(2, _NBLK),
    in_specs=[
        pl.BlockSpec((_BLK, 1), lambda p, i: (i * (1 - p), 0)),
        pl.BlockSpec((_BLK, _D), lambda p, i: (i * (1 - p), 0)),
        pl.BlockSpec((_NC, _BLK, _D), lambda p, i: (0, i * (1 - p), 0)),
        pl.BlockSpec((_D, _D), lambda p, i: (0, 0)),
        pl.BlockSpec((1, _D), lambda p, i: (0, 0)),
        pl.BlockSpec((1, _D), lambda p, i: (0, 0)),
        pl.BlockSpec((1, _D), lambda p, i: (0, 0)),
        pl.BlockSpec((_D, _T), lambda p, i: (0, 0)),
        pl.BlockSpec((1, _T), lambda p, i: (0, 0)),
    ],
    out_specs=pl.BlockSpec((1, _T), lambda p, i: (0, 0)),
    out_shape=jax.ShapeDtypeStruct((1, _T), jnp.float32),
    scratch_shapes=[
        pltpu.VMEM((2, _D), jnp.float32),
        pltpu.VMEM((_N, _D), jnp.float32),
        pltpu.VMEM((1, _D), jnp.float32),
    ],
)


def kernel(x, adjacency_list, degree_list,
           W1s, W1n, b1, g1, be1, W2s, W2n, b2, g2, be2, Wd, bd):
    src = adjacency_list[0].reshape(_NW, _NCH, 1, _CHB)
    dst = adjacency_list[1].reshape(_NW, _NCH, 1, _CHB)
    deg = degree_list.reshape(_N, 1)

    sc_agg = _build_sc_agg()
    xs = _tc_mm(x, W1s)
    agg1 = sc_agg(x, src, dst)
    h1 = _tc_layer1(deg, xs, agg1, W1n, b1.reshape(1, _D),
                    g1.reshape(1, _D), be1.reshape(1, _D))

    hs2 = _tc_mm(h1, W2s)
    agg2 = sc_agg(h1, src, dst)
    out = _tc_layer2(deg, hs2, agg2, W2n, b2.reshape(1, _D),
                     g2.reshape(1, _D), be2.reshape(1, _D),
                     Wd, bd.reshape(1, _T))
    return out.reshape(_T)
